# baseline (speedup 1.0000x reference)
import functools

import jax
import jax.numpy as jnp
from jax.experimental import pallas as pl
from jax.experimental.pallas import tpu as pltpu


def _fused_kernel(adj_ref, x_ref, w1_ref, b1_ref, b2_ref, g_ref, bt_ref,
                  w2_ref, fc3t_ref, fc3b_ref, o_ref, s_ref, *, block_m):
    p = pl.program_id(0)
    i = pl.program_id(1)

    @pl.when((p == 0) & (i == 0))
    def _prologue():
        s_ref[0, :x_ref.shape[0], :] = jnp.dot(
            x_ref[:], w1_ref[:], preferred_element_type=jnp.float32)

    n = adj_ref.shape[1]
    y = jnp.dot(adj_ref[:], s_ref[p, :n, :], preferred_element_type=jnp.float32)
    b = jnp.where(p == 0, b1_ref[:], b2_ref[:])
    h = y + b
    mean = jnp.mean(h, axis=1, keepdims=True)
    var = jnp.mean((h - mean) ** 2, axis=1, keepdims=True)
    h = (h - mean) * jax.lax.rsqrt(var + 1e-5)
    h = h * g_ref[:] + bt_ref[:]
    h = jnp.where(h >= 0, h, 0.01 * h)
    w = jnp.where(p == 0, w2_ref[:], fc3t_ref[:])
    tail = jnp.dot(h, w, preferred_element_type=jnp.float32)

    @pl.when(p == 0)
    def _store_s1():
        s_ref[1, pl.ds(i * block_m, block_m), :] = tail

    @pl.when(p == 1)
    def _store_out():
        o_ref[:] = tail + fc3b_ref[:]


def kernel(x, adjx, W1, b1, W2, b2, gn_gamma, gn_beta, fc3_W, fc3_b):
    n, d_in = x.shape
    d_h = W1.shape[1]
    d_out = fc3_W.shape[0]
    block_m = 560
    num_blocks = pl.cdiv(n, block_m)
    n_pad = num_blocks * block_m

    row = lambda v: v.reshape(1, -1)
    const = lambda shape: pl.BlockSpec(shape, lambda p, i: (0,) * len(shape))

    return pl.pallas_call(
        functools.partial(_fused_kernel, block_m=block_m),
        grid=(2, num_blocks),
        in_specs=[
            pl.BlockSpec((block_m, n), lambda p, i: (i, 0)),
            const((n, d_in)),
            const((d_in, d_h)),
            const((1, d_h)),
            const((1, d_h)),
            const((1, d_h)),
            const((1, d_h)),
            const((d_h, d_h)),
            const((d_h, d_out)),
            const((1, d_out)),
        ],
        out_specs=pl.BlockSpec((block_m, d_out), lambda p, i: (i * p, 0)),
        out_shape=jax.ShapeDtypeStruct((n, d_out), jnp.float32),
        scratch_shapes=[pltpu.VMEM((2, n_pad, d_h), jnp.float32)],
        compiler_params=pltpu.CompilerParams(
            dimension_semantics=("arbitrary", "arbitrary"),
            vmem_limit_bytes=64 * 1024 * 1024,
        ),
    )(adjx, x, W1, row(b1), row(b2), row(gn_gamma), row(gn_beta), W2,
      fc3_W.T, row(fc3_b))


# speedup vs baseline: 1.0105x; 1.0105x over previous
import functools

import jax
import jax.numpy as jnp
from jax.experimental import pallas as pl
from jax.experimental.pallas import tpu as pltpu


def _fused_kernel(adj_ref, x_ref, w1_ref, b1_ref, b2_ref, g_ref, bt_ref,
                  w2_ref, fc3t_ref, fc3b_ref, o_ref, s_ref, *, block_m):
    p = pl.program_id(0)
    i = pl.program_id(1)

    @pl.when((p == 0) & (i == 0))
    def _prologue():
        s_ref[0] = jnp.dot(x_ref[:], w1_ref[:],
                           preferred_element_type=jnp.float32)

    y = jnp.dot(adj_ref[:], s_ref[p], preferred_element_type=jnp.float32)
    b = jnp.where(p == 0, b1_ref[:], b2_ref[:])
    h = y + b
    mean = jnp.mean(h, axis=1, keepdims=True)
    var = jnp.mean((h - mean) ** 2, axis=1, keepdims=True)
    h = (h - mean) * jax.lax.rsqrt(var + 1e-5)
    h = h * g_ref[:] + bt_ref[:]
    h = jnp.where(h >= 0, h, 0.01 * h)
    w = jnp.where(p == 0, w2_ref[:], fc3t_ref[:])
    tail = jnp.dot(h, w, preferred_element_type=jnp.float32)

    @pl.when(p == 0)
    def _store_s1():
        s_ref[1, pl.ds(i * block_m, block_m), :] = tail

    @pl.when(p == 1)
    def _store_out():
        o_ref[:] = tail + fc3b_ref[:]


def kernel(x, adjx, W1, b1, W2, b2, gn_gamma, gn_beta, fc3_W, fc3_b):
    n, d_in = x.shape
    d_h = W1.shape[1]
    d_out = fc3_W.shape[0]
    block_m = 400

    row = lambda v: v.reshape(1, -1)
    const = lambda shape: pl.BlockSpec(shape, lambda p, i: (0,) * len(shape))

    return pl.pallas_call(
        functools.partial(_fused_kernel, block_m=block_m),
        grid=(2, n // block_m),
        in_specs=[
            pl.BlockSpec((block_m, n), lambda p, i: (i, 0)),
            const((n, d_in)),
            const((d_in, d_h)),
            const((1, d_h)),
            const((1, d_h)),
            const((1, d_h)),
            const((1, d_h)),
            const((d_h, d_h)),
            const((d_h, d_out)),
            const((1, d_out)),
        ],
        out_specs=pl.BlockSpec((block_m, d_out), lambda p, i: (i * p, 0)),
        out_shape=jax.ShapeDtypeStruct((n, d_out), jnp.float32),
        scratch_shapes=[pltpu.VMEM((2, n, d_h), jnp.float32)],
        compiler_params=pltpu.CompilerParams(
            dimension_semantics=("arbitrary", "arbitrary"),
        ),
    )(adjx, x, W1, row(b1), row(b2), row(gn_gamma), row(gn_beta), W2,
      fc3_W.T, row(fc3_b))
